# baseline (speedup 1.0000x reference)
import jax
import jax.numpy as jnp
from jax.experimental import pallas as pl
from jax.experimental.pallas import tpu as pltpu

B, S, D = 4, 4096, 4096


def _pool_body(mask_ref, hid_ref, out_ref, sems):
    totals = jnp.sum(mask_ref[...], axis=1)
    lp = jnp.sum(mask_ref[:, pl.ds(S - 1, 1)])
    copies = []
    for b in range(B):
        idx = jnp.where(lp == B, S - 1, (totals[b] - 1) & (S - 1))
        copies.append(pltpu.make_async_copy(
            hid_ref.at[b, idx], out_ref.at[b], sems.at[b]))
    for c in copies:
        c.start()
    for c in copies:
        c.wait()


def kernel(last_hidden_state, attention_mask):
    return pl.pallas_call(
        _pool_body,
        in_specs=[
            pl.BlockSpec((B, S), lambda: (0, 0)),
            pl.BlockSpec(memory_space=pl.ANY),
        ],
        out_specs=pl.BlockSpec(memory_space=pl.ANY),
        out_shape=jax.ShapeDtypeStruct((B, D), jnp.float32),
        scratch_shapes=[pltpu.SemaphoreType.DMA((B,))],
    )(attention_mask.astype(jnp.int32), last_hidden_state)


# speedup vs baseline: 1.6359x; 1.6359x over previous
import jax
import jax.numpy as jnp
from jax.experimental import pallas as pl
from jax.experimental.pallas import tpu as pltpu

B, S, D = 4, 4096, 4096


def _pool_body(mask_ref, hid_ref, out_ref, sems):
    totals = jnp.sum(mask_ref[...], axis=1)
    lp = jnp.sum(mask_ref[:, pl.ds(S - 1, 1)])
    copies = []
    for b in range(B):
        idx = jnp.where(lp == B, S - 1, (totals[b] - 1) & (S - 1))
        copies.append(pltpu.make_async_copy(
            hid_ref.at[b, idx], out_ref.at[b], sems.at[b]))
    for c in copies:
        c.start()
    for c in copies:
        c.wait()


def kernel(last_hidden_state, attention_mask):
    return pl.pallas_call(
        _pool_body,
        in_specs=[
            pl.BlockSpec((B, S), lambda: (0, 0)),
            pl.BlockSpec(memory_space=pl.ANY),
        ],
        out_specs=pl.BlockSpec((B, D), lambda: (0, 0)),
        out_shape=jax.ShapeDtypeStruct((B, D), jnp.float32),
        scratch_shapes=[pltpu.SemaphoreType.DMA((B,))],
    )(attention_mask.astype(jnp.int32), last_hidden_state)
